# baseline (speedup 1.0000x reference)
import functools

import jax
import jax.numpy as jnp
from jax import lax
from jax.experimental import pallas as pl
from jax.experimental.pallas import tpu as pltpu
from jax.experimental.pallas import tpu_sc as plsc

N_NODES = 200
BINS = 50
LAST = 1.0
P = N_NODES * (N_NODES - 1) // 2
C = P * BINS
NEV = 1000000
T_TOTAL = NEV + C

_NC, _NS, _L = 2, 16, 16
_NW = _NC * _NS

_BE = 4000
_NCH_E = NEV // _BE

_B = 2048
_NCHUNK = 992
_TPAD = _NCHUNK * _B

_PPAD = 19904

_mesh = plsc.VectorSubcoreMesh(core_axis_name="c", subcore_axis_name="s")
_sc_params = pltpu.CompilerParams(needs_layout_passes=False)


def _bin_of(t, bl_v):
    m0 = jnp.minimum(jnp.maximum((t * 50.0).astype(jnp.int32), 0), BINS - 1)
    b0 = plsc.load_gather(bl_v, [m0])
    b1 = plsc.load_gather(bl_v, [m0 + 1])
    m = m0 - jnp.where(t < b0, 1, 0)
    m = m + jnp.where((t >= b1) & (m0 < BINS - 1), 1, 0)
    return m


def _event_pass_sc(row_g, row_gn, t_gx, t_gn, zc, bl64, bln64):
    out_type = (
        jax.ShapeDtypeStruct((_NC, C), jnp.int32),
        jax.ShapeDtypeStruct((NEV,), jnp.float32),
    )
    scratch = [
        pltpu.VMEM_SHARED((C,), jnp.int32),
        pltpu.VMEM((64,), jnp.float32),
        pltpu.VMEM((64,), jnp.float32),
        pltpu.VMEM((_BE,), jnp.int32),
        pltpu.VMEM((_BE,), jnp.int32),
        pltpu.VMEM((_BE,), jnp.float32),
        pltpu.VMEM((_BE,), jnp.float32),
        pltpu.VMEM((_BE,), jnp.int32),
        pltpu.VMEM((_BE,), jnp.int32),
        pltpu.VMEM((_BE,), jnp.float32),
    ]

    @functools.partial(pl.kernel, mesh=_mesh, out_type=out_type,
                       scratch_types=scratch, compiler_params=_sc_params)
    def k(rg_h, rgn_h, tg_h, tgn_h, zc_h, bl_h, bln_h,
          hist_h, delta_h,
          spmem, bl_v, bln_v, row_v, rown_v, t_v, tn_v,
          kq_v, ones_v, dval_v):
        cid = lax.axis_index("c")
        sid = lax.axis_index("s")
        wid = sid * _NC + cid
        lane = lax.iota(jnp.int32, _L)
        pltpu.sync_copy(bl_h, bl_v)
        pltpu.sync_copy(bln_h, bln_v)

        @pl.when(sid == 0)
        def _():
            pltpu.sync_copy(zc_h, spmem)

        def fill_ones(i, c2):
            ones_v[pl.ds(i * _L, _L)] = jnp.ones((_L,), jnp.int32)
            return c2
        lax.fori_loop(0, _BE // _L, fill_ones, 0)
        plsc.subcore_barrier()

        nt = (_NCH_E - 1 - wid) // _NW + 1

        def chunk_body(tt, carry):
            base = (wid + tt * _NW) * _BE
            pltpu.sync_copy(rg_h.at[pl.ds(base, _BE)], row_v)
            pltpu.sync_copy(rgn_h.at[pl.ds(base, _BE)], rown_v)
            pltpu.sync_copy(tg_h.at[pl.ds(base, _BE)], t_v)
            pltpu.sync_copy(tgn_h.at[pl.ds(base, _BE)], tn_v)

            def vec_body(vi, c2):
                sl = pl.ds(vi * _L, _L)
                r = row_v[sl]
                rn = rown_v[sl]
                t = t_v[sl]
                tn = tn_v[sl]
                m = _bin_of(t, bl_v)
                mn = _bin_of(tn, bl_v)
                kq = r * BINS + m
                kqn = rn * BINS + mn
                end = kq != kqn
                bln = plsc.load_gather(bln_v, [m])
                delta = jnp.where(end, bln, tn) - t
                kq_v[sl] = kq
                dval_v[sl] = delta
                return c2

            lax.fori_loop(0, _BE // _L, vec_body, 0)
            pltpu.sync_copy(ones_v, spmem.at[kq_v], add=True)
            pltpu.sync_copy(dval_v, delta_h.at[pl.ds(base, _BE)])
            return carry

        lax.fori_loop(0, nt, chunk_body, 0)
        plsc.subcore_barrier()

        @pl.when(sid == 0)
        def _():
            pltpu.sync_copy(spmem, hist_h.at[cid])

    return k(row_g, row_gn, t_gx, t_gn, zc, bl64, bln64)


def _border_body(cum_ref, m_ref, h_ref, bl_ref, bln_ref,
                 st_ref, dl_ref, tb_ref):
    cm = cum_ref[...]
    st_ref[...] = (cm - cm[:, 0:1]) & 1
    blv = bl_ref[...]
    dl_ref[...] = jnp.where(h_ref[...] > 0, m_ref[...], bln_ref[...]) - blv
    tb_ref[...] = jnp.broadcast_to(blv, cm.shape)


def _border_tc(cum_pad2, m2, h2, bl2, bln2):
    grid = 8
    rows = _PPAD // grid
    return pl.pallas_call(
        _border_body,
        grid=(grid,),
        in_specs=[
            pl.BlockSpec((rows, BINS), lambda g: (g, 0)),
            pl.BlockSpec((rows, BINS), lambda g: (g, 0)),
            pl.BlockSpec((rows, BINS), lambda g: (g, 0)),
            pl.BlockSpec((1, BINS), lambda g: (0, 0)),
            pl.BlockSpec((1, BINS), lambda g: (0, 0)),
        ],
        out_specs=[
            pl.BlockSpec((rows, BINS), lambda g: (g, 0)),
            pl.BlockSpec((rows, BINS), lambda g: (g, 0)),
            pl.BlockSpec((rows, BINS), lambda g: (g, 0)),
        ],
        out_shape=[
            jax.ShapeDtypeStruct((_PPAD, BINS), jnp.int32),
            jax.ShapeDtypeStruct((_PPAD, BINS), jnp.float32),
            jax.ShapeDtypeStruct((_PPAD, BINS), jnp.float32),
        ],
    )(cum_pad2, m2, h2, bl2, bln2)


def _assemble_sc(t_cat, state_cat, delta_cat, starts_pad):
    out_type = (
        jax.ShapeDtypeStruct((_TPAD,), jnp.float32),
        jax.ShapeDtypeStruct((_TPAD,), jnp.int32),
        jax.ShapeDtypeStruct((_TPAD,), jnp.int32),
        jax.ShapeDtypeStruct((_TPAD,), jnp.float32),
    )
    scratch = [
        pltpu.VMEM((128,), jnp.int32),
        pltpu.VMEM((_B,), jnp.int32),
        pltpu.VMEM((_B,), jnp.int32),
        pltpu.VMEM((_B,), jnp.float32),
        pltpu.VMEM((_B,), jnp.int32),
        pltpu.VMEM((_B,), jnp.float32),
        pltpu.SemaphoreType.DMA,
        pltpu.SemaphoreType.DMA,
        pltpu.SemaphoreType.DMA,
    ]

    @functools.partial(pl.kernel, mesh=_mesh, out_type=out_type,
                       scratch_types=scratch, compiler_params=_sc_params)
    def k(tc_hbm, sc_hbm, dc_hbm, st_hbm, t_out, ev_out, s_out, d_out,
          starts_v, idx_v, ev_v, tg_v, sg_v, dg_v, sem1, sem2, sem3):
        wid = lax.axis_index("s") * _NC + lax.axis_index("c")
        pltpu.sync_copy(st_hbm, starts_v)
        lane = lax.iota(jnp.int32, _L)

        def chunk_body(tt, carry):
            base = (wid + tt * _NW) * _B

            def vec_body(vi, c2):
                q = base + vi * _L + lane
                pos = jnp.zeros((_L,), jnp.int32)
                for s in (64, 32, 16, 8, 4, 2, 1):
                    cand = pos + s
                    sv = plsc.load_gather(starts_v, [cand])
                    pos = jnp.where(sv <= q, cand, pos)
                sstart = plsc.load_gather(starts_v, [pos])
                kreg = lax.shift_right_logical(pos, 1)
                is_bd = (pos & 1) == 0
                idx_bd = (q - sstart) * BINS + kreg
                idx_ev = C + q - (kreg + 1) * P
                idx = jnp.where(is_bd, idx_bd, idx_ev)
                idx = jnp.minimum(jnp.maximum(idx, 0), T_TOTAL - 1)
                idx = jnp.where(q < T_TOTAL, idx, q - (_TPAD - T_TOTAL))
                idx_v[pl.ds(vi * _L, _L)] = idx
                ev_v[pl.ds(vi * _L, _L)] = jnp.where(
                    is_bd, jnp.zeros((_L,), jnp.int32),
                    jnp.ones((_L,), jnp.int32))
                return c2

            lax.fori_loop(0, _B // _L, vec_body, 0)
            cp1 = pltpu.async_copy(tc_hbm.at[idx_v], tg_v, sem1)
            cp2 = pltpu.async_copy(sc_hbm.at[idx_v], sg_v, sem2)
            cp3 = pltpu.async_copy(dc_hbm.at[idx_v], dg_v, sem3)
            cp1.wait()
            cp2.wait()
            cp3.wait()
            pltpu.sync_copy(tg_v, t_out.at[pl.ds(base, _B)])
            pltpu.sync_copy(ev_v, ev_out.at[pl.ds(base, _B)])
            pltpu.sync_copy(sg_v, s_out.at[pl.ds(base, _B)])
            pltpu.sync_copy(dg_v, d_out.at[pl.ds(base, _B)])
            return carry

        lax.fori_loop(0, _NCHUNK // _NW, chunk_body, 0)

    return k(t_cat, state_cat, delta_cat, starts_pad)


def kernel(pairs, times):
    n = N_NODES
    i = pairs[0].astype(jnp.int32)
    j = pairs[1].astype(jnp.int32)
    rows = i * (2 * n - i - 1) // 2 + (j - i - 1)
    bl = jnp.linspace(0.0, LAST, BINS + 1)[:-1].astype(jnp.float32)
    blnext = jnp.concatenate([bl[1:], jnp.full((1,), LAST, jnp.float32)])
    nev = times.shape[0]

    ts, row_s = lax.sort((times, rows), num_keys=1, is_stable=True)
    e_cnt = jnp.searchsorted(ts, bl, side='left').astype(jnp.int32)

    iota = jnp.arange(nev, dtype=jnp.int32)
    row_g, e_g = lax.sort((row_s, iota), num_keys=1, is_stable=True)
    _, ginv = lax.sort((e_g, iota), num_keys=1, is_stable=True)
    t_gx = ts[e_g]
    row_gn = jnp.concatenate([row_g[1:], jnp.full((1,), -1, jnp.int32)])
    t_gn = jnp.concatenate([t_gx[1:], jnp.zeros((1,), jnp.float32)])

    bl64 = jnp.full((64,), 2.0, jnp.float32).at[:BINS].set(bl)
    bln64 = jnp.full((64,), 2.0, jnp.float32).at[:BINS].set(blnext)
    zc = jnp.zeros((C,), jnp.int32)

    hist2, delta_g = _event_pass_sc(
        row_g, row_gn, t_gx, t_gn, zc, bl64, bln64)
    h = hist2[0] + hist2[1]
    cum = jnp.cumsum(h)
    cum_pad = jnp.concatenate([jnp.zeros((1,), jnp.int32), cum[:-1]])
    rs_arr = cum_pad[0::BINS]

    delta_ev = delta_g[ginv]
    state_ev = ((ginv - rs_arr[row_s] + 1) & 1).astype(jnp.int32)

    m_first = t_gx[jnp.minimum(cum_pad, nev - 1)]

    pad_flat = _PPAD * BINS
    cum2 = jnp.zeros((pad_flat,), jnp.int32).at[:C].set(cum_pad)
    m2 = jnp.zeros((pad_flat,), jnp.float32).at[:C].set(m_first)
    h2 = jnp.zeros((pad_flat,), jnp.int32).at[:C].set(h)
    state_bd, delta_bd, t_bd = _border_tc(
        cum2.reshape(_PPAD, BINS), m2.reshape(_PPAD, BINS),
        h2.reshape(_PPAD, BINS), bl.reshape(1, BINS), blnext.reshape(1, BINS))

    t_cat = jnp.concatenate([t_bd.reshape(-1)[:C], ts])
    state_cat = jnp.concatenate([state_bd.reshape(-1)[:C], state_ev])
    delta_cat = jnp.concatenate([delta_bd.reshape(-1)[:C], delta_ev])
    k_arr = jnp.arange(BINS, dtype=jnp.int32)
    bstart = k_arr * P + e_cnt
    estart = (k_arr + 1) * P + e_cnt
    starts = jnp.stack([bstart, estart], axis=1).reshape(-1)
    starts_pad = jnp.full((128,), _TPAD, jnp.int32).at[:2 * BINS].set(starts)

    t_o, ev_o, s_o, d_o = _assemble_sc(t_cat, state_cat, delta_cat, starts_pad)
    return (t_o[:T_TOTAL], ev_o[:T_TOTAL].astype(bool), s_o[:T_TOTAL],
            d_o[:T_TOTAL])


# speedup vs baseline: 3.6313x; 3.6313x over previous
import functools

import jax
import jax.numpy as jnp
from jax import lax
from jax.experimental import pallas as pl
from jax.experimental.pallas import tpu as pltpu
from jax.experimental.pallas import tpu_sc as plsc

N_NODES = 200
BINS = 50
LAST = 1.0
P = N_NODES * (N_NODES - 1) // 2
C = P * BINS
NEV = 1000000
T_TOTAL = NEV + C

_NC, _NS, _L = 2, 16, 16
_NW = _NC * _NS

_BE = 4000
_NCH_E = NEV // _BE

_B = 2048
_NCHUNK = 992
_TPAD = _NCHUNK * _B

_PPAD = 19904

_BP = 4000
_NCH_P = NEV // _BP
_HALF = NEV // 2
_BC = 4000
_NCH_C = 249
_CPAD = _NCH_C * _BC

_mesh = plsc.VectorSubcoreMesh(core_axis_name="c", subcore_axis_name="s")
_sc_params = pltpu.CompilerParams(needs_layout_passes=False)


def _bin_of(t, bl_v):
    m0 = jnp.minimum(jnp.maximum((t * 50.0).astype(jnp.int32), 0), BINS - 1)
    b0 = plsc.load_gather(bl_v, [m0])
    b1 = plsc.load_gather(bl_v, [m0 + 1])
    m = m0 - jnp.where(t < b0, 1, 0)
    m = m + jnp.where((t >= b1) & (m0 < BINS - 1), 1, 0)
    return m


def _event_pass_sc(row_g, row_gn, t_gx, t_gn, zc, bl64, bln64):
    out_type = (
        jax.ShapeDtypeStruct((_NC, C), jnp.int32),
        jax.ShapeDtypeStruct((NEV,), jnp.float32),
    )
    scratch = [
        pltpu.VMEM_SHARED((C,), jnp.int32),
        pltpu.VMEM((64,), jnp.float32),
        pltpu.VMEM((64,), jnp.float32),
        pltpu.VMEM((_BE,), jnp.int32),
        pltpu.VMEM((_BE,), jnp.int32),
        pltpu.VMEM((_BE,), jnp.float32),
        pltpu.VMEM((_BE,), jnp.float32),
        pltpu.VMEM((_BE,), jnp.int32),
        pltpu.VMEM((_BE,), jnp.int32),
        pltpu.VMEM((_BE,), jnp.float32),
    ]

    @functools.partial(pl.kernel, mesh=_mesh, out_type=out_type,
                       scratch_types=scratch, compiler_params=_sc_params)
    def k(rg_h, rgn_h, tg_h, tgn_h, zc_h, bl_h, bln_h,
          hist_h, delta_h,
          spmem, bl_v, bln_v, row_v, rown_v, t_v, tn_v,
          kq_v, ones_v, dval_v):
        cid = lax.axis_index("c")
        sid = lax.axis_index("s")
        wid = sid * _NC + cid
        lane = lax.iota(jnp.int32, _L)
        pltpu.sync_copy(bl_h, bl_v)
        pltpu.sync_copy(bln_h, bln_v)

        @pl.when(sid == 0)
        def _():
            pltpu.sync_copy(zc_h, spmem)

        def fill_ones(i, c2):
            ones_v[pl.ds(i * _L, _L)] = jnp.ones((_L,), jnp.int32)
            return c2
        lax.fori_loop(0, _BE // _L, fill_ones, 0)
        plsc.subcore_barrier()

        nt = (_NCH_E - 1 - wid) // _NW + 1

        def chunk_body(tt, carry):
            base = (wid + tt * _NW) * _BE
            pltpu.sync_copy(rg_h.at[pl.ds(base, _BE)], row_v)
            pltpu.sync_copy(rgn_h.at[pl.ds(base, _BE)], rown_v)
            pltpu.sync_copy(tg_h.at[pl.ds(base, _BE)], t_v)
            pltpu.sync_copy(tgn_h.at[pl.ds(base, _BE)], tn_v)

            def vec_body(vi, c2):
                sl = pl.ds(vi * _L, _L)
                r = row_v[sl]
                rn = rown_v[sl]
                t = t_v[sl]
                tn = tn_v[sl]
                m = _bin_of(t, bl_v)
                mn = _bin_of(tn, bl_v)
                kq = r * BINS + m
                kqn = rn * BINS + mn
                end = kq != kqn
                bln = plsc.load_gather(bln_v, [m])
                delta = jnp.where(end, bln, tn) - t
                kq_v[sl] = kq
                dval_v[sl] = delta
                return c2

            lax.fori_loop(0, _BE // _L, vec_body, 0)
            pltpu.sync_copy(ones_v, spmem.at[kq_v], add=True)
            pltpu.sync_copy(dval_v, delta_h.at[pl.ds(base, _BE)])
            return carry

        lax.fori_loop(0, nt, chunk_body, 0)
        plsc.subcore_barrier()

        @pl.when(sid == 0)
        def _():
            pltpu.sync_copy(spmem, hist_h.at[cid])

    return k(row_g, row_gn, t_gx, t_gn, zc, bl64, bln64)


def _post_pass_sc(e_g, delta_g, row_g, rs_pad, t_gx, cum_c):
    out_type = (
        jax.ShapeDtypeStruct((NEV,), jnp.float32),
        jax.ShapeDtypeStruct((NEV,), jnp.float32),
        jax.ShapeDtypeStruct((_CPAD,), jnp.float32),
    )
    scratch = [
        pltpu.VMEM_SHARED((NEV,), jnp.float32),
        pltpu.VMEM((19904,), jnp.int32),
        pltpu.VMEM((_BP,), jnp.int32),
        pltpu.VMEM((_BP,), jnp.float32),
        pltpu.VMEM((_BP,), jnp.int32),
        pltpu.VMEM((_BP,), jnp.float32),
        pltpu.VMEM((_BC,), jnp.int32),
        pltpu.VMEM((_BC,), jnp.float32),
    ]

    @functools.partial(pl.kernel, mesh=_mesh, out_type=out_type,
                       scratch_types=scratch, compiler_params=_sc_params)
    def k(eg_h, dg_h, rg_h, rs_h, tg_h, cum_h, dev_h, sev_h, m_h,
          spmem, rs_v, eg_v, dv_v, row_v, st_v, cum_v, mb_v):
        cid = lax.axis_index("c")
        sid = lax.axis_index("s")
        wid = sid * _NC + cid
        lane = lax.iota(jnp.int32, _L)
        pltpu.sync_copy(rs_h, rs_v)
        nt1 = (_NCH_P - 1 - sid) // _NS + 1

        def p1(tt, carry):
            base = (sid + tt * _NS) * _BP
            pltpu.sync_copy(eg_h.at[pl.ds(base, _BP)], eg_v)
            pltpu.sync_copy(dg_h.at[pl.ds(base, _BP)], dv_v)
            pltpu.sync_copy(dv_v, spmem.at[eg_v])
            return carry

        lax.fori_loop(0, nt1, p1, 0)
        plsc.subcore_barrier()

        @pl.when((sid == 0) & (cid == 0))
        def _():
            pltpu.sync_copy(spmem, dev_h)

        plsc.subcore_barrier()

        def p2(tt, carry):
            base = (sid + tt * _NS) * _BP
            pltpu.sync_copy(eg_h.at[pl.ds(base, _BP)], eg_v)
            pltpu.sync_copy(rg_h.at[pl.ds(base, _BP)], row_v)

            def v2(vi, c2):
                sl = pl.ds(vi * _L, _L)
                g = base + vi * _L + lane
                rs = plsc.load_gather(rs_v, [row_v[sl]])
                st_v[sl] = ((g - rs + 1) & 1).astype(jnp.float32)
                return c2

            lax.fori_loop(0, _BP // _L, v2, 0)
            pltpu.sync_copy(st_v, spmem.at[eg_v])
            return carry

        lax.fori_loop(0, nt1, p2, 0)
        plsc.subcore_barrier()

        @pl.when((sid == 0) & (cid == 0))
        def _():
            pltpu.sync_copy(spmem, sev_h)

        plsc.subcore_barrier()

        @pl.when(sid == 0)
        def _():
            pltpu.sync_copy(tg_h, spmem)

        plsc.subcore_barrier()
        nt3 = (_NCH_C - 1 - wid) // _NW + 1

        def p3(tt, carry):
            base = (wid + tt * _NW) * _BC
            pltpu.sync_copy(cum_h.at[pl.ds(base, _BC)], cum_v)
            pltpu.sync_copy(spmem.at[cum_v], mb_v)
            pltpu.sync_copy(mb_v, m_h.at[pl.ds(base, _BC)])
            return carry

        lax.fori_loop(0, nt3, p3, 0)

    return k(e_g, delta_g, row_g, rs_pad, t_gx, cum_c)


def _border_body(cum_ref, m_ref, h_ref, bl_ref, bln_ref,
                 st_ref, dl_ref, tb_ref):
    cm = cum_ref[...]
    st_ref[...] = (cm - cm[:, 0:1]) & 1
    blv = bl_ref[...]
    dl_ref[...] = jnp.where(h_ref[...] > 0, m_ref[...], bln_ref[...]) - blv
    tb_ref[...] = jnp.broadcast_to(blv, cm.shape)


def _border_tc(cum_pad2, m2, h2, bl2, bln2):
    grid = 8
    rows = _PPAD // grid
    return pl.pallas_call(
        _border_body,
        grid=(grid,),
        in_specs=[
            pl.BlockSpec((rows, BINS), lambda g: (g, 0)),
            pl.BlockSpec((rows, BINS), lambda g: (g, 0)),
            pl.BlockSpec((rows, BINS), lambda g: (g, 0)),
            pl.BlockSpec((1, BINS), lambda g: (0, 0)),
            pl.BlockSpec((1, BINS), lambda g: (0, 0)),
        ],
        out_specs=[
            pl.BlockSpec((rows, BINS), lambda g: (g, 0)),
            pl.BlockSpec((rows, BINS), lambda g: (g, 0)),
            pl.BlockSpec((rows, BINS), lambda g: (g, 0)),
        ],
        out_shape=[
            jax.ShapeDtypeStruct((_PPAD, BINS), jnp.int32),
            jax.ShapeDtypeStruct((_PPAD, BINS), jnp.float32),
            jax.ShapeDtypeStruct((_PPAD, BINS), jnp.float32),
        ],
    )(cum_pad2, m2, h2, bl2, bln2)


def _assemble_sc(t_cat, state_cat, delta_cat, starts_pad):
    out_type = (
        jax.ShapeDtypeStruct((_TPAD,), jnp.float32),
        jax.ShapeDtypeStruct((_TPAD,), jnp.int32),
        jax.ShapeDtypeStruct((_TPAD,), jnp.int32),
        jax.ShapeDtypeStruct((_TPAD,), jnp.float32),
    )
    scratch = [
        pltpu.VMEM((128,), jnp.int32),
        pltpu.VMEM((_B,), jnp.int32),
        pltpu.VMEM((_B,), jnp.int32),
        pltpu.VMEM((_B,), jnp.float32),
        pltpu.VMEM((_B,), jnp.int32),
        pltpu.VMEM((_B,), jnp.float32),
        pltpu.SemaphoreType.DMA,
        pltpu.SemaphoreType.DMA,
        pltpu.SemaphoreType.DMA,
    ]

    @functools.partial(pl.kernel, mesh=_mesh, out_type=out_type,
                       scratch_types=scratch, compiler_params=_sc_params)
    def k(tc_hbm, sc_hbm, dc_hbm, st_hbm, t_out, ev_out, s_out, d_out,
          starts_v, idx_v, ev_v, tg_v, sg_v, dg_v, sem1, sem2, sem3):
        wid = lax.axis_index("s") * _NC + lax.axis_index("c")
        pltpu.sync_copy(st_hbm, starts_v)
        lane = lax.iota(jnp.int32, _L)

        def chunk_body(tt, carry):
            base = (wid + tt * _NW) * _B

            def vec_body(vi, c2):
                q = base + vi * _L + lane
                pos = jnp.zeros((_L,), jnp.int32)
                for s in (64, 32, 16, 8, 4, 2, 1):
                    cand = pos + s
                    sv = plsc.load_gather(starts_v, [cand])
                    pos = jnp.where(sv <= q, cand, pos)
                sstart = plsc.load_gather(starts_v, [pos])
                kreg = lax.shift_right_logical(pos, 1)
                is_bd = (pos & 1) == 0
                idx_bd = (q - sstart) * BINS + kreg
                idx_ev = C + q - (kreg + 1) * P
                idx = jnp.where(is_bd, idx_bd, idx_ev)
                idx = jnp.minimum(jnp.maximum(idx, 0), T_TOTAL - 1)
                idx = jnp.where(q < T_TOTAL, idx, q - (_TPAD - T_TOTAL))
                idx_v[pl.ds(vi * _L, _L)] = idx
                ev_v[pl.ds(vi * _L, _L)] = jnp.where(
                    is_bd, jnp.zeros((_L,), jnp.int32),
                    jnp.ones((_L,), jnp.int32))
                return c2

            lax.fori_loop(0, _B // _L, vec_body, 0)
            cp1 = pltpu.async_copy(tc_hbm.at[idx_v], tg_v, sem1)
            cp2 = pltpu.async_copy(sc_hbm.at[idx_v], sg_v, sem2)
            cp3 = pltpu.async_copy(dc_hbm.at[idx_v], dg_v, sem3)
            cp1.wait()
            cp2.wait()
            cp3.wait()
            pltpu.sync_copy(tg_v, t_out.at[pl.ds(base, _B)])
            pltpu.sync_copy(ev_v, ev_out.at[pl.ds(base, _B)])
            pltpu.sync_copy(sg_v, s_out.at[pl.ds(base, _B)])
            pltpu.sync_copy(dg_v, d_out.at[pl.ds(base, _B)])
            return carry

        lax.fori_loop(0, _NCHUNK // _NW, chunk_body, 0)

    return k(t_cat, state_cat, delta_cat, starts_pad)


def kernel(pairs, times):
    n = N_NODES
    i = pairs[0].astype(jnp.int32)
    j = pairs[1].astype(jnp.int32)
    rows = i * (2 * n - i - 1) // 2 + (j - i - 1)
    bl = jnp.linspace(0.0, LAST, BINS + 1)[:-1].astype(jnp.float32)
    blnext = jnp.concatenate([bl[1:], jnp.full((1,), LAST, jnp.float32)])
    nev = times.shape[0]

    ts, row_s = lax.sort((times, rows), num_keys=1, is_stable=True)
    e_cnt = jnp.searchsorted(ts, bl, side='left').astype(jnp.int32)

    iota = jnp.arange(nev, dtype=jnp.int32)
    row_g, e_g = lax.sort((row_s, iota), num_keys=1, is_stable=True)
    t_gx = ts[e_g]
    row_gn = jnp.concatenate([row_g[1:], jnp.full((1,), -1, jnp.int32)])
    t_gn = jnp.concatenate([t_gx[1:], jnp.zeros((1,), jnp.float32)])

    bl64 = jnp.full((64,), 2.0, jnp.float32).at[:BINS].set(bl)
    bln64 = jnp.full((64,), 2.0, jnp.float32).at[:BINS].set(blnext)
    zc = jnp.zeros((C,), jnp.int32)

    hist2, delta_g = _event_pass_sc(
        row_g, row_gn, t_gx, t_gn, zc, bl64, bln64)
    h = hist2[0] + hist2[1]
    cum = jnp.cumsum(h)
    cum_pad = jnp.concatenate([jnp.zeros((1,), jnp.int32), cum[:-1]])
    rs_arr = cum_pad[0::BINS]

    rs_pad = jnp.zeros((19904,), jnp.int32).at[:P].set(rs_arr)
    cum_c = jnp.zeros((_CPAD,), jnp.int32).at[:C].set(
        jnp.minimum(cum_pad, nev - 1))
    delta_ev, state_f, m_full = _post_pass_sc(
        e_g, delta_g, row_g, rs_pad, t_gx, cum_c)
    state_ev = state_f.astype(jnp.int32)
    m_first = m_full[:C]

    pad_flat = _PPAD * BINS
    cum2 = jnp.zeros((pad_flat,), jnp.int32).at[:C].set(cum_pad)
    m2 = jnp.zeros((pad_flat,), jnp.float32).at[:C].set(m_first)
    h2 = jnp.zeros((pad_flat,), jnp.int32).at[:C].set(h)
    state_bd, delta_bd, t_bd = _border_tc(
        cum2.reshape(_PPAD, BINS), m2.reshape(_PPAD, BINS),
        h2.reshape(_PPAD, BINS), bl.reshape(1, BINS), blnext.reshape(1, BINS))

    t_cat = jnp.concatenate([t_bd.reshape(-1)[:C], ts])
    state_cat = jnp.concatenate([state_bd.reshape(-1)[:C], state_ev])
    delta_cat = jnp.concatenate([delta_bd.reshape(-1)[:C], delta_ev])
    k_arr = jnp.arange(BINS, dtype=jnp.int32)
    bstart = k_arr * P + e_cnt
    estart = (k_arr + 1) * P + e_cnt
    starts = jnp.stack([bstart, estart], axis=1).reshape(-1)
    starts_pad = jnp.full((128,), _TPAD, jnp.int32).at[:2 * BINS].set(starts)

    t_o, ev_o, s_o, d_o = _assemble_sc(t_cat, state_cat, delta_cat, starts_pad)
    return (t_o[:T_TOTAL], ev_o[:T_TOTAL].astype(bool), s_o[:T_TOTAL],
            d_o[:T_TOTAL])


# speedup vs baseline: 3.6347x; 1.0009x over previous
import functools

import jax
import jax.numpy as jnp
from jax import lax
from jax.experimental import pallas as pl
from jax.experimental.pallas import tpu as pltpu
from jax.experimental.pallas import tpu_sc as plsc

N_NODES = 200
BINS = 50
LAST = 1.0
P = N_NODES * (N_NODES - 1) // 2
C = P * BINS
NEV = 1000000
T_TOTAL = NEV + C

_NC, _NS, _L = 2, 16, 16
_NW = _NC * _NS

_BE = 4000
_NCH_E = NEV // _BE

_B = 2048
_NCHUNK = 992
_TPAD = _NCHUNK * _B

_PPAD = 19904

_BP = 4000
_NCH_P = NEV // _BP
_BC = 4000
_NCH_C = 249
_CPAD = _NCH_C * _BC

_mesh = plsc.VectorSubcoreMesh(core_axis_name="c", subcore_axis_name="s")
_sc_params = pltpu.CompilerParams(needs_layout_passes=False)


def _bin_of(t, bl_v):
    m0 = jnp.minimum(jnp.maximum((t * 50.0).astype(jnp.int32), 0), BINS - 1)
    b0 = plsc.load_gather(bl_v, [m0])
    b1 = plsc.load_gather(bl_v, [m0 + 1])
    m = m0 - jnp.where(t < b0, 1, 0)
    m = m + jnp.where((t >= b1) & (m0 < BINS - 1), 1, 0)
    return m


def _event_pass_sc(row_g, row_gn, t_gx, t_gn, zc, bl64, bln64):
    out_type = (
        jax.ShapeDtypeStruct((_NC, C), jnp.int32),
        jax.ShapeDtypeStruct((NEV,), jnp.float32),
    )
    scratch = [
        pltpu.VMEM_SHARED((C,), jnp.int32),
        pltpu.VMEM((64,), jnp.float32),
        pltpu.VMEM((64,), jnp.float32),
        pltpu.VMEM((_BE,), jnp.int32),
        pltpu.VMEM((_BE,), jnp.int32),
        pltpu.VMEM((_BE,), jnp.float32),
        pltpu.VMEM((_BE,), jnp.float32),
        pltpu.VMEM((_BE,), jnp.int32),
        pltpu.VMEM((_BE,), jnp.int32),
        pltpu.VMEM((_BE,), jnp.float32),
    ]

    @functools.partial(pl.kernel, mesh=_mesh, out_type=out_type,
                       scratch_types=scratch, compiler_params=_sc_params)
    def k(rg_h, rgn_h, tg_h, tgn_h, zc_h, bl_h, bln_h,
          hist_h, delta_h,
          spmem, bl_v, bln_v, row_v, rown_v, t_v, tn_v,
          kq_v, ones_v, dval_v):
        cid = lax.axis_index("c")
        sid = lax.axis_index("s")
        wid = sid * _NC + cid
        lane = lax.iota(jnp.int32, _L)
        pltpu.sync_copy(bl_h, bl_v)
        pltpu.sync_copy(bln_h, bln_v)

        @pl.when(sid == 0)
        def _():
            pltpu.sync_copy(zc_h, spmem)

        def fill_ones(i, c2):
            ones_v[pl.ds(i * _L, _L)] = jnp.ones((_L,), jnp.int32)
            return c2
        lax.fori_loop(0, _BE // _L, fill_ones, 0)
        plsc.subcore_barrier()

        nt = (_NCH_E - 1 - wid) // _NW + 1

        def chunk_body(tt, carry):
            base = (wid + tt * _NW) * _BE
            pltpu.sync_copy(rg_h.at[pl.ds(base, _BE)], row_v)
            pltpu.sync_copy(rgn_h.at[pl.ds(base, _BE)], rown_v)
            pltpu.sync_copy(tg_h.at[pl.ds(base, _BE)], t_v)
            pltpu.sync_copy(tgn_h.at[pl.ds(base, _BE)], tn_v)

            def vec_body(vi, c2):
                sl = pl.ds(vi * _L, _L)
                r = row_v[sl]
                rn = rown_v[sl]
                t = t_v[sl]
                tn = tn_v[sl]
                m = _bin_of(t, bl_v)
                mn = _bin_of(tn, bl_v)
                kq = r * BINS + m
                kqn = rn * BINS + mn
                end = kq != kqn
                bln = plsc.load_gather(bln_v, [m])
                delta = jnp.where(end, bln, tn) - t
                kq_v[sl] = kq
                dval_v[sl] = delta
                return c2

            lax.fori_loop(0, _BE // _L, vec_body, 0)
            pltpu.sync_copy(ones_v, spmem.at[kq_v], add=True)
            pltpu.sync_copy(dval_v, delta_h.at[pl.ds(base, _BE)])
            return carry

        lax.fori_loop(0, nt, chunk_body, 0)
        plsc.subcore_barrier()

        @pl.when(sid == 0)
        def _():
            pltpu.sync_copy(spmem, hist_h.at[cid])

    return k(row_g, row_gn, t_gx, t_gn, zc, bl64, bln64)


def _post_pass_sc(e_g, delta_g, row_g, rs_pad, t_gx, cum_c):
    out_type = (
        jax.ShapeDtypeStruct((NEV,), jnp.float32),
        jax.ShapeDtypeStruct((NEV,), jnp.float32),
        jax.ShapeDtypeStruct((_CPAD,), jnp.float32),
    )
    scratch = [
        pltpu.VMEM_SHARED((NEV,), jnp.float32),
        pltpu.VMEM((19904,), jnp.int32),
        pltpu.VMEM((_BP,), jnp.int32),
        pltpu.VMEM((_BP,), jnp.float32),
        pltpu.VMEM((_BP,), jnp.int32),
        pltpu.VMEM((_BP,), jnp.float32),
        pltpu.VMEM((_BC,), jnp.int32),
        pltpu.VMEM((_BC,), jnp.float32),
    ]

    @functools.partial(pl.kernel, mesh=_mesh, out_type=out_type,
                       scratch_types=scratch, compiler_params=_sc_params)
    def k(eg_h, dg_h, rg_h, rs_h, tg_h, cum_h, dev_h, sev_h, m_h,
          spmem, rs_v, eg_v, dv_v, row_v, st_v, cum_v, mb_v):
        cid = lax.axis_index("c")
        sid = lax.axis_index("s")
        wid = sid * _NC + cid
        lane = lax.iota(jnp.int32, _L)
        pltpu.sync_copy(rs_h, rs_v)
        nt1 = (_NCH_P - 1 - sid) // _NS + 1

        def p1(tt, carry):
            base = (sid + tt * _NS) * _BP
            pltpu.sync_copy(eg_h.at[pl.ds(base, _BP)], eg_v)
            pltpu.sync_copy(dg_h.at[pl.ds(base, _BP)], dv_v)
            pltpu.sync_copy(dv_v, spmem.at[eg_v])
            return carry

        lax.fori_loop(0, nt1, p1, 0)
        plsc.subcore_barrier()

        @pl.when((sid == 0) & (cid == 0))
        def _():
            pltpu.sync_copy(spmem, dev_h)

        plsc.subcore_barrier()

        def p2(tt, carry):
            base = (sid + tt * _NS) * _BP
            pltpu.sync_copy(eg_h.at[pl.ds(base, _BP)], eg_v)
            pltpu.sync_copy(rg_h.at[pl.ds(base, _BP)], row_v)

            def v2(vi, c2):
                sl = pl.ds(vi * _L, _L)
                g = base + vi * _L + lane
                rs = plsc.load_gather(rs_v, [row_v[sl]])
                st_v[sl] = ((g - rs + 1) & 1).astype(jnp.float32)
                return c2

            lax.fori_loop(0, _BP // _L, v2, 0)
            pltpu.sync_copy(st_v, spmem.at[eg_v])
            return carry

        lax.fori_loop(0, nt1, p2, 0)
        plsc.subcore_barrier()

        @pl.when((sid == 0) & (cid == 0))
        def _():
            pltpu.sync_copy(spmem, sev_h)

        plsc.subcore_barrier()

        @pl.when(sid == 0)
        def _():
            pltpu.sync_copy(tg_h, spmem)

        plsc.subcore_barrier()
        nt3 = (_NCH_C - 1 - wid) // _NW + 1

        def p3(tt, carry):
            base = (wid + tt * _NW) * _BC
            pltpu.sync_copy(cum_h.at[pl.ds(base, _BC)], cum_v)
            pltpu.sync_copy(spmem.at[cum_v], mb_v)
            pltpu.sync_copy(mb_v, m_h.at[pl.ds(base, _BC)])
            return carry

        lax.fori_loop(0, nt3, p3, 0)

    return k(e_g, delta_g, row_g, rs_pad, t_gx, cum_c)


def _border_body(cum_ref, m_ref, h_ref, bl_ref, bln_ref,
                 st_ref, dl_ref, tb_ref):
    cm = cum_ref[...]
    st_ref[...] = (cm - cm[:, 0:1]) & 1
    blv = bl_ref[...]
    dl_ref[...] = jnp.where(h_ref[...] > 0, m_ref[...], bln_ref[...]) - blv
    tb_ref[...] = jnp.broadcast_to(blv, cm.shape)


def _border_tc(cum_pad2, m2, h2, bl2, bln2):
    grid = 8
    rows = _PPAD // grid
    return pl.pallas_call(
        _border_body,
        grid=(grid,),
        in_specs=[
            pl.BlockSpec((rows, BINS), lambda g: (g, 0)),
            pl.BlockSpec((rows, BINS), lambda g: (g, 0)),
            pl.BlockSpec((rows, BINS), lambda g: (g, 0)),
            pl.BlockSpec((1, BINS), lambda g: (0, 0)),
            pl.BlockSpec((1, BINS), lambda g: (0, 0)),
        ],
        out_specs=[
            pl.BlockSpec((rows, BINS), lambda g: (g, 0)),
            pl.BlockSpec((rows, BINS), lambda g: (g, 0)),
            pl.BlockSpec((rows, BINS), lambda g: (g, 0)),
        ],
        out_shape=[
            jax.ShapeDtypeStruct((_PPAD, BINS), jnp.int32),
            jax.ShapeDtypeStruct((_PPAD, BINS), jnp.float32),
            jax.ShapeDtypeStruct((_PPAD, BINS), jnp.float32),
        ],
    )(cum_pad2, m2, h2, bl2, bln2)


def _assemble_sc(t_cat, state_cat, delta_cat, starts_pad):
    out_type = (
        jax.ShapeDtypeStruct((_TPAD,), jnp.float32),
        jax.ShapeDtypeStruct((_TPAD,), jnp.int32),
        jax.ShapeDtypeStruct((_TPAD,), jnp.int32),
        jax.ShapeDtypeStruct((_TPAD,), jnp.float32),
    )
    scratch = [
        pltpu.VMEM((128,), jnp.int32),
        pltpu.VMEM((_B,), jnp.int32),
        pltpu.VMEM((_B,), jnp.int32),
        pltpu.VMEM((_B,), jnp.float32),
        pltpu.VMEM((_B,), jnp.int32),
        pltpu.VMEM((_B,), jnp.float32),
        pltpu.SemaphoreType.DMA,
        pltpu.SemaphoreType.DMA,
        pltpu.SemaphoreType.DMA,
    ]

    @functools.partial(pl.kernel, mesh=_mesh, out_type=out_type,
                       scratch_types=scratch, compiler_params=_sc_params)
    def k(tc_hbm, sc_hbm, dc_hbm, st_hbm, t_out, ev_out, s_out, d_out,
          starts_v, idx_v, ev_v, tg_v, sg_v, dg_v, sem1, sem2, sem3):
        wid = lax.axis_index("s") * _NC + lax.axis_index("c")
        pltpu.sync_copy(st_hbm, starts_v)
        lane = lax.iota(jnp.int32, _L)

        def chunk_body(tt, carry):
            base = (wid + tt * _NW) * _B

            def vec_body(vi, c2):
                q = base + vi * _L + lane
                pos = jnp.zeros((_L,), jnp.int32)
                for s in (64, 32, 16, 8, 4, 2, 1):
                    cand = pos + s
                    sv = plsc.load_gather(starts_v, [cand])
                    pos = jnp.where(sv <= q, cand, pos)
                sstart = plsc.load_gather(starts_v, [pos])
                kreg = lax.shift_right_logical(pos, 1)
                is_bd = (pos & 1) == 0
                idx_bd = (q - sstart) * BINS + kreg
                idx_ev = C + q - (kreg + 1) * P
                idx = jnp.where(is_bd, idx_bd, idx_ev)
                idx = jnp.minimum(jnp.maximum(idx, 0), T_TOTAL - 1)
                idx = jnp.where(q < T_TOTAL, idx, q - (_TPAD - T_TOTAL))
                idx_v[pl.ds(vi * _L, _L)] = idx
                ev_v[pl.ds(vi * _L, _L)] = jnp.where(
                    is_bd, jnp.zeros((_L,), jnp.int32),
                    jnp.ones((_L,), jnp.int32))
                return c2

            lax.fori_loop(0, _B // _L, vec_body, 0)
            cp1 = pltpu.async_copy(tc_hbm.at[idx_v], tg_v, sem1)
            cp2 = pltpu.async_copy(sc_hbm.at[idx_v], sg_v, sem2)
            cp3 = pltpu.async_copy(dc_hbm.at[idx_v], dg_v, sem3)
            cp1.wait()
            cp2.wait()
            cp3.wait()
            pltpu.sync_copy(tg_v, t_out.at[pl.ds(base, _B)])
            pltpu.sync_copy(ev_v, ev_out.at[pl.ds(base, _B)])
            pltpu.sync_copy(sg_v, s_out.at[pl.ds(base, _B)])
            pltpu.sync_copy(dg_v, d_out.at[pl.ds(base, _B)])
            return carry

        lax.fori_loop(0, _NCHUNK // _NW, chunk_body, 0)

    return k(t_cat, state_cat, delta_cat, starts_pad)


def kernel(pairs, times):
    n = N_NODES
    i = pairs[0].astype(jnp.int32)
    j = pairs[1].astype(jnp.int32)
    rows = i * (2 * n - i - 1) // 2 + (j - i - 1)
    bl = jnp.linspace(0.0, LAST, BINS + 1)[:-1].astype(jnp.float32)
    blnext = jnp.concatenate([bl[1:], jnp.full((1,), LAST, jnp.float32)])
    nev = times.shape[0]

    ts, row_s = lax.sort((times, rows), num_keys=1, is_stable=True)
    e_cnt = jnp.searchsorted(ts, bl, side='left').astype(jnp.int32)

    iota = jnp.arange(nev, dtype=jnp.int32)
    row_g, e_g = lax.sort((row_s, iota), num_keys=1, is_stable=True)
    t_gx = ts[e_g]
    row_gn = jnp.concatenate([row_g[1:], jnp.full((1,), -1, jnp.int32)])
    t_gn = jnp.concatenate([t_gx[1:], jnp.zeros((1,), jnp.float32)])

    bl64 = jnp.full((64,), 2.0, jnp.float32).at[:BINS].set(bl)
    bln64 = jnp.full((64,), 2.0, jnp.float32).at[:BINS].set(blnext)
    zc = jnp.zeros((C,), jnp.int32)

    hist2, delta_g = _event_pass_sc(
        row_g, row_gn, t_gx, t_gn, zc, bl64, bln64)
    h = hist2[0] + hist2[1]
    cum = jnp.cumsum(h)
    cum_pad = jnp.concatenate([jnp.zeros((1,), jnp.int32), cum[:-1]])
    rs_arr = cum_pad[0::BINS]

    rs_pad = jnp.zeros((19904,), jnp.int32).at[:P].set(rs_arr)
    cum_c = jnp.zeros((_CPAD,), jnp.int32).at[:C].set(
        jnp.minimum(cum_pad, nev - 1))
    delta_ev, state_f, m_full = _post_pass_sc(
        e_g, delta_g, row_g, rs_pad, t_gx, cum_c)
    state_ev = state_f.astype(jnp.int32)
    m_first = m_full[:C]

    pad_flat = _PPAD * BINS
    cum2 = jnp.zeros((pad_flat,), jnp.int32).at[:C].set(cum_pad)
    m2 = jnp.zeros((pad_flat,), jnp.float32).at[:C].set(m_first)
    h2 = jnp.zeros((pad_flat,), jnp.int32).at[:C].set(h)
    state_bd, delta_bd, t_bd = _border_tc(
        cum2.reshape(_PPAD, BINS), m2.reshape(_PPAD, BINS),
        h2.reshape(_PPAD, BINS), bl.reshape(1, BINS), blnext.reshape(1, BINS))

    t_cat = jnp.concatenate([t_bd.reshape(-1)[:C], ts])
    state_cat = jnp.concatenate([state_bd.reshape(-1)[:C], state_ev])
    delta_cat = jnp.concatenate([delta_bd.reshape(-1)[:C], delta_ev])
    k_arr = jnp.arange(BINS, dtype=jnp.int32)
    bstart = k_arr * P + e_cnt
    estart = (k_arr + 1) * P + e_cnt
    starts = jnp.stack([bstart, estart], axis=1).reshape(-1)
    starts_pad = jnp.full((128,), _TPAD, jnp.int32).at[:2 * BINS].set(starts)

    t_o, ev_o, s_o, d_o = _assemble_sc(t_cat, state_cat, delta_cat, starts_pad)
    return (t_o[:T_TOTAL], ev_o[:T_TOTAL].astype(bool), s_o[:T_TOTAL],
            d_o[:T_TOTAL])
